# baseline (speedup 1.0000x reference)
import functools

import jax
import jax.numpy as jnp
from jax import lax
from jax.experimental import pallas as pl
from jax.experimental.pallas import tpu as pltpu
from jax.experimental.pallas import tpu_sc as plsc

_K = 5
_TOPK = _K + 1
_RUN = 128
_BIG_IDX = 2**30


def _knn_body(n_valid, num_tiles, tn, f_ref, m_ref, idx_ref, vals, idxs):
    i = pl.program_id(0)
    b = f_ref.shape[0]

    @pl.when(i == 0)
    def _init():
        vals[...] = jnp.full(vals.shape, jnp.inf, jnp.float32)
        idxs[...] = jnp.full(idxs.shape, _BIG_IDX, jnp.int32)

    f = f_ref[...]
    norm = jnp.sqrt(jnp.sum(f * f, axis=1, keepdims=True))
    fn = f / jnp.clip(norm, 1e-12, None)
    q_sq = jnp.sum(fn * fn, axis=1, keepdims=True)

    m = m_ref[...]
    ones_row = jnp.ones((1, m.shape[1]), jnp.float32)
    m_sq = lax.dot_general(ones_row, m * m, (((1,), (1,)), ((), ())),
                           preferred_element_type=jnp.float32)
    s = lax.dot_general(fn, m, (((1,), (1,)), ((), ())),
                        precision=lax.Precision.HIGHEST,
                        preferred_element_type=jnp.float32)
    dist = (q_sq - 2.0 * s) + m_sq

    gcol = i * tn + lax.broadcasted_iota(jnp.int32, (1, tn), 1)
    dist = jnp.where(gcol < n_valid, dist, jnp.inf)

    cur_v = jnp.concatenate([vals[...], dist], axis=1)
    cur_i = jnp.concatenate(
        [idxs[...], jnp.broadcast_to(gcol, (b, tn))], axis=1)

    new_v, new_i = [], []
    for _ in range(_TOPK):
        v = jnp.min(cur_v, axis=1, keepdims=True)
        pick = jnp.min(jnp.where(cur_v == v, cur_i, _BIG_IDX),
                       axis=1, keepdims=True)
        new_v.append(v)
        new_i.append(pick)
        cur_v = jnp.where(cur_i == pick, jnp.inf, cur_v)
    vals[:, 0:_TOPK] = jnp.concatenate(new_v, axis=1)
    idxs[:, 0:_TOPK] = jnp.concatenate(new_i, axis=1)

    @pl.when(i == num_tiles - 1)
    def _emit():
        idx_ref[...] = idxs[:, 0:8]


def _topk_indices(features, feat_memory, tn=2048):
    b, d = features.shape
    n = feat_memory.shape[0]
    num_tiles = (n + tn - 1) // tn
    n_pad = num_tiles * tn
    if n_pad != n:
        feat_memory = jnp.pad(feat_memory, ((0, n_pad - n), (0, 0)))
    grid = (num_tiles,)
    return pl.pallas_call(
        functools.partial(_knn_body, n, num_tiles, tn),
        grid=grid,
        in_specs=[
            pl.BlockSpec((b, d), lambda i: (0, 0)),
            pl.BlockSpec((tn, d), lambda i: (i, 0)),
        ],
        out_specs=pl.BlockSpec((b, 8), lambda i: (0, 0)),
        out_shape=jax.ShapeDtypeStruct((b, 8), jnp.int32),
        scratch_shapes=[
            pltpu.VMEM((b, _RUN), jnp.float32),
            pltpu.VMEM((b, _RUN), jnp.int32),
        ],
    )(features, feat_memory)


_CPAD = 128
_NEG = -3.4e38


def _gather_mean_argmax(pred_pad, flat_idx, num_classes):
    n, cpad = pred_pad.shape
    bq = flat_idx.shape[0] // _K
    info = plsc.get_sparse_core_info()
    nc = info.num_cores
    nw = nc * info.num_subcores
    qpw = bq // nw
    ipw = qpw * _K
    half = ipw // 2
    qph = qpw // 2
    nslice = cpad // 16

    mesh = plsc.VectorSubcoreMesh(core_axis_name="c", subcore_axis_name="s")

    @functools.partial(
        pl.kernel, mesh=mesh,
        out_type=[jax.ShapeDtypeStruct((bq, cpad), jnp.float32),
                  jax.ShapeDtypeStruct((bq,), jnp.int32)],
        scratch_types=[
            pltpu.VMEM((half,), jnp.int32),
            pltpu.VMEM((half,), jnp.int32),
            pltpu.VMEM((half, cpad), jnp.float32),
            pltpu.VMEM((half, cpad), jnp.float32),
            pltpu.VMEM((qpw, cpad), jnp.float32),
            pltpu.VMEM((qpw,), jnp.int32),
            pltpu.SemaphoreType.DMA,
        ],
    )
    def sc_kernel(pred_hbm, idx_hbm, logits_hbm, labels_hbm,
                  idx_a, idx_b, rows_a, rows_b, log_v, lab_v, sem):
        wid = lax.axis_index("s") * nc + lax.axis_index("c")
        base = wid * ipw
        pltpu.sync_copy(idx_hbm.at[pl.ds(base, half)], idx_a)
        pltpu.sync_copy(idx_hbm.at[pl.ds(base + half, half)], idx_b)
        cp_a = pltpu.async_copy(pred_hbm.at[idx_a], rows_a, sem)
        cp_b = pltpu.async_copy(pred_hbm.at[idx_b], rows_b, sem)
        cp_a.wait()
        cp_b.wait()

        lane = lax.iota(jnp.int32, 16)

        def _all_lanes(v, op):
            for sh in (8, 4, 2, 1):
                v = op(v, v.at[lane ^ sh].get(mode="promise_in_bounds"))
            return v

        def make_body(rows, qoff):
            def body(q, labvec):
                accs = []
                for j in range(nslice):
                    a = rows[q * _K, pl.ds(j * 16, 16)]
                    for r in range(1, _K):
                        a = a + rows[q * _K + r, pl.ds(j * 16, 16)]
                    a = a * (1.0 / _K)
                    log_v[qoff + q, pl.ds(j * 16, 16)] = a
                    accs.append(a)
                gmax = None
                masked = []
                for j, a in enumerate(accs):
                    av = jnp.where(lane + j * 16 < num_classes, a, _NEG)
                    masked.append(av)
                    gmax = av if gmax is None else jnp.maximum(gmax, av)
                gmax = _all_lanes(gmax, jnp.maximum)
                pos = None
                for j, av in enumerate(masked):
                    cand = jnp.where(av == gmax, lane + j * 16, _BIG_IDX)
                    pos = cand if pos is None else jnp.minimum(pos, cand)
                pos = _all_lanes(pos, jnp.minimum)
                return jnp.where(lane == q, pos, labvec)
            return body

        zeros16 = jnp.zeros((16,), jnp.int32)
        lab_v[pl.ds(0, qph)] = lax.fori_loop(
            0, qph, make_body(rows_a, 0), zeros16)
        lab_v[pl.ds(qph, qph)] = lax.fori_loop(
            0, qph, make_body(rows_b, qph), zeros16)

        pltpu.sync_copy(log_v, logits_hbm.at[pl.ds(wid * qpw, qpw)])
        pltpu.sync_copy(lab_v, labels_hbm.at[pl.ds(wid * qpw, qpw)])

    return sc_kernel(pred_pad, flat_idx)


def kernel(features, feat_memory, pred_memory):
    c = pred_memory.shape[1]
    idx8 = _topk_indices(features, feat_memory)
    neigh = idx8[:, 1:1 + _K].reshape(-1)
    pred_pad = jnp.pad(pred_memory, ((0, 0), (0, _CPAD - c)))
    logits_pad, pseudo_labels = _gather_mean_argmax(pred_pad, neigh, c)
    return (pseudo_labels, logits_pad[:, :c])


# speedup vs baseline: 1.1147x; 1.1147x over previous
import functools

import jax
import jax.numpy as jnp
from jax import lax
from jax.experimental import pallas as pl
from jax.experimental.pallas import tpu as pltpu
from jax.experimental.pallas import tpu_sc as plsc

_K = 5
_TOPK = _K + 1
_RUN = 128
_BIG_IDX = 2**30


def _knn_body(n_valid, num_tiles, tn, f_ref, m_ref, idx_ref, vals, idxs):
    i = pl.program_id(0)
    b = f_ref.shape[0]

    @pl.when(i == 0)
    def _init():
        vals[...] = jnp.full(vals.shape, jnp.inf, jnp.float32)
        idxs[...] = jnp.full(idxs.shape, float(_BIG_IDX), jnp.float32)

    f = f_ref[...]
    norm = jnp.sqrt(jnp.sum(f * f, axis=1, keepdims=True))
    fn = f / jnp.clip(norm, 1e-12, None)
    q_sq = jnp.sum(fn * fn, axis=1, keepdims=True)

    m = m_ref[...]
    ones_row = jnp.ones((1, m.shape[1]), jnp.float32)
    m_sq = lax.dot_general(ones_row, m * m, (((1,), (1,)), ((), ())),
                           preferred_element_type=jnp.float32)
    s = lax.dot_general(fn, m, (((1,), (1,)), ((), ())),
                        precision=lax.Precision.HIGHEST,
                        preferred_element_type=jnp.float32)
    dist = (q_sq - 2.0 * s) + m_sq

    gcol = i * tn + lax.broadcasted_iota(jnp.int32, (1, tn), 1)
    dist = jnp.where(gcol < n_valid, dist, jnp.inf)
    gcolf = gcol.astype(jnp.float32)

    tile_v, tile_i = [], []
    cur = dist
    for _ in range(_TOPK):
        v = jnp.min(cur, axis=1, keepdims=True)
        pick = jnp.min(jnp.where(cur == v, gcolf, float(_BIG_IDX)),
                       axis=1, keepdims=True)
        tile_v.append(v)
        tile_i.append(pick)
        cur = jnp.where(gcolf == pick, jnp.inf, cur)
    cand_v = jnp.concatenate([vals[:, 0:_TOPK]] + tile_v, axis=1)
    cand_i = jnp.concatenate([idxs[:, 0:_TOPK]] + tile_i, axis=1)
    new_v, new_i = [], []
    for _ in range(_TOPK):
        v = jnp.min(cand_v, axis=1, keepdims=True)
        pick = jnp.min(jnp.where(cand_v == v, cand_i, float(_BIG_IDX)),
                       axis=1, keepdims=True)
        new_v.append(v)
        new_i.append(pick)
        cand_v = jnp.where(cand_i == pick, jnp.inf, cand_v)
    vals[:, 0:_TOPK] = jnp.concatenate(new_v, axis=1)
    idxs[:, 0:_TOPK] = jnp.concatenate(new_i, axis=1)

    @pl.when(i == num_tiles - 1)
    def _emit():
        idx_ref[...] = idxs[:, 0:8].astype(jnp.int32)


def _topk_indices(features, feat_memory, tn=4096):
    b, d = features.shape
    n = feat_memory.shape[0]
    num_tiles = (n + tn - 1) // tn
    n_pad = num_tiles * tn
    if n_pad != n:
        feat_memory = jnp.pad(feat_memory, ((0, n_pad - n), (0, 0)))
    grid = (num_tiles,)
    return pl.pallas_call(
        functools.partial(_knn_body, n, num_tiles, tn),
        grid=grid,
        in_specs=[
            pl.BlockSpec((b, d), lambda i: (0, 0)),
            pl.BlockSpec((tn, d), lambda i: (i, 0)),
        ],
        out_specs=pl.BlockSpec((b, 8), lambda i: (0, 0)),
        out_shape=jax.ShapeDtypeStruct((b, 8), jnp.int32),
        scratch_shapes=[
            pltpu.VMEM((b, _RUN), jnp.float32),
            pltpu.VMEM((b, _RUN), jnp.float32),
        ],
    )(features, feat_memory)


_CPAD = 128
_NEG = -3.4e38


def _gather_mean_argmax(pred_pad, flat_idx, num_classes):
    n, cpad = pred_pad.shape
    bq = flat_idx.shape[0] // _K
    info = plsc.get_sparse_core_info()
    nc = info.num_cores
    nw = nc * info.num_subcores
    qpw = bq // nw
    ipw = qpw * _K
    half = ipw // 2
    qph = qpw // 2
    nslice = cpad // 16

    mesh = plsc.VectorSubcoreMesh(core_axis_name="c", subcore_axis_name="s")

    @functools.partial(
        pl.kernel, mesh=mesh,
        out_type=[jax.ShapeDtypeStruct((bq, cpad), jnp.float32),
                  jax.ShapeDtypeStruct((bq,), jnp.int32)],
        scratch_types=[
            pltpu.VMEM((half,), jnp.int32),
            pltpu.VMEM((half,), jnp.int32),
            pltpu.VMEM((half, cpad), jnp.float32),
            pltpu.VMEM((half, cpad), jnp.float32),
            pltpu.VMEM((qpw, cpad), jnp.float32),
            pltpu.VMEM((qpw,), jnp.int32),
            pltpu.SemaphoreType.DMA,
        ],
    )
    def sc_kernel(pred_hbm, idx_hbm, logits_hbm, labels_hbm,
                  idx_a, idx_b, rows_a, rows_b, log_v, lab_v, sem):
        wid = lax.axis_index("s") * nc + lax.axis_index("c")
        base = wid * ipw
        pltpu.sync_copy(idx_hbm.at[pl.ds(base, half)], idx_a)
        pltpu.sync_copy(idx_hbm.at[pl.ds(base + half, half)], idx_b)
        cp_a = pltpu.async_copy(pred_hbm.at[idx_a], rows_a, sem)
        cp_b = pltpu.async_copy(pred_hbm.at[idx_b], rows_b, sem)
        cp_a.wait()
        cp_b.wait()

        lane = lax.iota(jnp.int32, 16)

        def _all_lanes(v, op):
            for sh in (8, 4, 2, 1):
                v = op(v, v.at[lane ^ sh].get(mode="promise_in_bounds"))
            return v

        def make_body(rows, qoff):
            def body(q, labvec):
                accs = []
                for j in range(nslice):
                    a = rows[q * _K, pl.ds(j * 16, 16)]
                    for r in range(1, _K):
                        a = a + rows[q * _K + r, pl.ds(j * 16, 16)]
                    a = a * (1.0 / _K)
                    log_v[qoff + q, pl.ds(j * 16, 16)] = a
                    accs.append(a)
                gmax = None
                masked = []
                for j, a in enumerate(accs):
                    av = jnp.where(lane + j * 16 < num_classes, a, _NEG)
                    masked.append(av)
                    gmax = av if gmax is None else jnp.maximum(gmax, av)
                gmax = _all_lanes(gmax, jnp.maximum)
                pos = None
                for j, av in enumerate(masked):
                    cand = jnp.where(av == gmax, lane + j * 16, _BIG_IDX)
                    pos = cand if pos is None else jnp.minimum(pos, cand)
                pos = _all_lanes(pos, jnp.minimum)
                return jnp.where(lane == q, pos, labvec)
            return body

        zeros16 = jnp.zeros((16,), jnp.int32)
        lab_v[pl.ds(0, qph)] = lax.fori_loop(
            0, qph, make_body(rows_a, 0), zeros16)
        lab_v[pl.ds(qph, qph)] = lax.fori_loop(
            0, qph, make_body(rows_b, qph), zeros16)

        pltpu.sync_copy(log_v, logits_hbm.at[pl.ds(wid * qpw, qpw)])
        pltpu.sync_copy(lab_v, labels_hbm.at[pl.ds(wid * qpw, qpw)])

    return sc_kernel(pred_pad, flat_idx)


def kernel(features, feat_memory, pred_memory):
    c = pred_memory.shape[1]
    idx8 = _topk_indices(features, feat_memory)
    neigh = idx8[:, 1:1 + _K].reshape(-1)
    pred_pad = jnp.pad(pred_memory, ((0, 0), (0, _CPAD - c)))
    logits_pad, pseudo_labels = _gather_mean_argmax(pred_pad, neigh, c)
    return (pseudo_labels, logits_pad[:, :c])
